# baseline (speedup 1.0000x reference)
import functools

import jax
import jax.numpy as jnp
from jax import lax
from jax.experimental import pallas as pl
from jax.experimental.pallas import tpu as pltpu
from jax.experimental.pallas import tpu_sc as plsc

NC = 2
NS = 16
NW = NC * NS
NB = 2
NI = 4


def _sc_aggregate(feat, src1, dst1, n_pad, chunk):
    n, d = feat.shape
    ew_pad = src1.shape[0] // NW
    n_iter = ew_pad // chunk
    rows_per_tile = n_pad // NS
    zcopies = rows_per_tile // chunk

    mesh = plsc.VectorSubcoreMesh(core_axis_name="c", subcore_axis_name="s")

    @functools.partial(
        pl.kernel,
        mesh=mesh,
        out_type=(
            jax.ShapeDtypeStruct((NC * n_pad, d), jnp.float32),
            jax.ShapeDtypeStruct((NC * n_pad,), jnp.float32),
        ),
        scratch_types=(
            [pltpu.VMEM((chunk, d), jnp.float32) for _ in range(NB)]
            + [pltpu.VMEM((chunk,), jnp.int32) for _ in range(NI)]
            + [pltpu.VMEM((chunk,), jnp.int32) for _ in range(NI)]
            + [
                pltpu.VMEM((chunk,), jnp.int32),
                pltpu.VMEM((chunk,), jnp.float32),
                pltpu.VMEM((rows_per_tile,), jnp.float32),
            ]
            + [pltpu.SemaphoreType.DMA for _ in range(NB)]
            + [pltpu.SemaphoreType.DMA for _ in range(NI)]
            + [
                pltpu.SemaphoreType.DMA,
                pltpu.SemaphoreType.DMA,
                pltpu.VMEM_SHARED((n_pad, d), jnp.float32),
                pltpu.VMEM_SHARED((n_pad,), jnp.float32),
            ]
        ),
    )
    def sc_kernel(feat_hbm, src_hbm, dst_hbm, agg_out, deg_out,
                  r0, r1, s0, s1, s2, s3, t0, t1, t2, t3, didx_s, ones_v,
                  dzero, g0, g1, i0, i1, i2, i3, ssem, dsem,
                  agg_sh, deg_sh):
        rows = [r0, r1]
        sidx = [s0, s1, s2, s3]
        didx = [t0, t1, t2, t3]
        gsem = [g0, g1]
        isem = [i0, i1, i2, i3]
        c = lax.axis_index("c")
        s = lax.axis_index("s")
        wid = s * NC + c
        base_r = s * rows_per_tile

        zeros16 = jnp.zeros((16,), jnp.float32)
        ones16 = jnp.ones((16,), jnp.float32)

        def zrow_body(i, _):
            for j in range(d // 16):
                r0[i, pl.ds(j * 16, 16)] = zeros16
            return _
        lax.fori_loop(0, chunk, zrow_body, None)

        def dz_body(i, _):
            dzero[pl.ds(i * 16, 16)] = zeros16
            return _
        lax.fori_loop(0, rows_per_tile // 16, dz_body, None)

        for i in range(chunk // 16):
            ones_v[pl.ds(i * 16, 16)] = ones16

        for k in range(zcopies):
            pltpu.sync_copy(r0, agg_sh.at[pl.ds(base_r + k * chunk, chunk)])
        pltpu.sync_copy(dzero, deg_sh.at[pl.ds(base_r, rows_per_tile)])
        plsc.subcore_barrier()

        ebase = wid * ew_pad

        def idx_start(j, sl):
            off = pl.multiple_of(ebase + j * chunk, 8)
            pltpu.async_copy(src_hbm.at[pl.ds(off, chunk)], sidx[sl],
                             isem[sl])
            pltpu.async_copy(dst_hbm.at[pl.ds(off, chunk)], didx[sl],
                             isem[sl])

        def idx_wait(j, sl):
            off = pl.multiple_of(ebase + j * chunk, 8)
            pltpu.make_async_copy(src_hbm.at[pl.ds(off, chunk)], sidx[sl],
                                  isem[sl]).wait()
            pltpu.make_async_copy(dst_hbm.at[pl.ds(off, chunk)], didx[sl],
                                  isem[sl]).wait()

        def gather_start(j, k):
            pltpu.async_copy(feat_hbm.at[sidx[k % NI]], rows[k % NB],
                             gsem[k % NB])

        def gather_wait(j, k):
            pltpu.make_async_copy(feat_hbm.at[sidx[k % NI]],
                                  rows[k % NB], gsem[k % NB]).wait()

        for k in range(NI):
            idx_start(k, k)
        for b in range(NB):
            idx_wait(b, b)
            gather_start(b, b)

        def group_body(i, _):
            for k in range(NI):
                j = i * NI + k
                b = k % NB
                gather_wait(j, k)
                for q in range(chunk // 16):
                    didx_s[pl.ds(q * 16, 16)] = didx[k][pl.ds(q * 16, 16)]
                pltpu.sync_copy(rows[b], agg_sh.at[didx_s], add=True)
                pltpu.sync_copy(ones_v, deg_sh.at[didx_s], add=True)
                idx_start(lax.rem(j + NI, n_iter), k)
                kn = (k + NB) % NI
                jn = lax.rem(j + NB, n_iter)
                idx_wait(jn, kn)
                gather_start(jn, kn)
            return _
        lax.fori_loop(0, n_iter // NI, group_body, None)

        for b in range(NB):
            gather_wait(b, b)
        for k in range(NB, NI):
            idx_wait(k, k)

        plsc.subcore_barrier()

        out_r = pl.multiple_of(c * n_pad + base_r, 8)
        pltpu.sync_copy(agg_sh.at[pl.ds(base_r, rows_per_tile)],
                        agg_out.at[pl.ds(out_r, rows_per_tile)])
        pltpu.sync_copy(deg_sh.at[pl.ds(base_r, rows_per_tile)],
                        deg_out.at[pl.ds(out_r, rows_per_tile)])

    return sc_kernel(feat, src1, dst1)


def _tc_combine(feat, agg_parts, deg_parts, w_self, w_neigh, b, blk):
    n, d = feat.shape
    d_out = w_self.shape[1]
    grid = n // blk
    deg3 = deg_parts[:, :, None]
    b2 = b[None, :]

    def body(feat_ref, agg_ref, deg_ref, ws_ref, wn_ref, b_ref, out_ref):
        agg = agg_ref[0] + agg_ref[1]
        deg = jnp.maximum(deg_ref[0] + deg_ref[1], 1.0)
        h = agg / deg
        out_ref[...] = (
            jnp.dot(feat_ref[...], ws_ref[...],
                    preferred_element_type=jnp.float32)
            + jnp.dot(h, wn_ref[...], preferred_element_type=jnp.float32)
            + b_ref[...]
        )

    return pl.pallas_call(
        body,
        grid=(grid,),
        in_specs=[
            pl.BlockSpec((blk, d), lambda i: (i, 0)),
            pl.BlockSpec((NC, blk, d), lambda i: (0, i, 0)),
            pl.BlockSpec((NC, blk, 1), lambda i: (0, i, 0)),
            pl.BlockSpec((d, d_out), lambda i: (0, 0)),
            pl.BlockSpec((d, d_out), lambda i: (0, 0)),
            pl.BlockSpec((1, d_out), lambda i: (0, 0)),
        ],
        out_specs=pl.BlockSpec((blk, d_out), lambda i: (i, 0)),
        out_shape=jax.ShapeDtypeStruct((n, d_out), jnp.float32),
    )(feat, agg_parts, deg3, w_self, w_neigh, b2)


def kernel(feat, edge_index, W_self, W_neigh, b):
    n, d = feat.shape
    e = edge_index.shape[1]
    chunk = 128
    ew = e // NW
    ew_pad = (-(-ew // (NI * chunk)) + 1) * (NI * chunk)
    n_iter = ew_pad // chunk
    n_pad = -(-n // (NS * chunk)) * (NS * chunk)
    src1 = jnp.pad(edge_index[0].reshape(NW, ew),
                   ((0, 0), (0, ew_pad - ew))).reshape(NW * ew_pad)
    dst1 = jnp.pad(edge_index[1].reshape(NW, ew),
                   ((0, 0), (0, ew_pad - ew)),
                   constant_values=n).reshape(NW * ew_pad)
    agg_flat, deg_flat = _sc_aggregate(feat, src1, dst1, n_pad, chunk)
    agg_parts = agg_flat.reshape(NC, n_pad, d)
    deg_parts = deg_flat.reshape(NC, n_pad)
    out = _tc_combine(feat, agg_parts, deg_parts, W_self, W_neigh, b, blk=2000)
    return out


# speedup vs baseline: 2.8969x; 2.8969x over previous
import functools

import jax
import jax.numpy as jnp
from jax import lax
from jax.experimental import pallas as pl
from jax.experimental.pallas import tpu as pltpu
from jax.experimental.pallas import tpu_sc as plsc

NC = 2
NS = 16
NW = NC * NS


def _sc_aggregate(feat, src, dst, n_pad, chunk):
    n, d = feat.shape
    e = src.shape[0]
    ew = e // NW
    n_iter = ew // chunk
    rows_per_tile = n_pad // NS
    zcopies = rows_per_tile // chunk

    mesh = plsc.VectorSubcoreMesh(core_axis_name="c", subcore_axis_name="s")

    @functools.partial(
        pl.kernel,
        mesh=mesh,
        out_type=(
            jax.ShapeDtypeStruct((NC * n_pad, d), jnp.float32),
            jax.ShapeDtypeStruct((NC * n_pad,), jnp.float32),
        ),
        scratch_types=[
            pltpu.VMEM((chunk,), jnp.int32),
            pltpu.VMEM((chunk,), jnp.int32),
            pltpu.VMEM((chunk, d), jnp.float32),
            pltpu.VMEM((chunk,), jnp.float32),
            pltpu.VMEM((rows_per_tile,), jnp.float32),
            pltpu.VMEM_SHARED((n_pad, d), jnp.float32),
            pltpu.VMEM_SHARED((n_pad,), jnp.float32),
            pltpu.SemaphoreType.DMA,
        ],
    )
    def sc_kernel(feat_hbm, src_hbm, dst_hbm, agg_out, deg_out,
                  src_idx, dst_idx, rows, ones_v, dzero, agg_sh, deg_sh, sem):
        c = lax.axis_index("c")
        s = lax.axis_index("s")
        wid = s * NC + c
        base_r = s * rows_per_tile

        zeros16 = jnp.zeros((16,), jnp.float32)
        ones16 = jnp.ones((16,), jnp.float32)

        def zrow_body(i, _):
            for j in range(d // 16):
                rows[i, pl.ds(j * 16, 16)] = zeros16
            return _
        lax.fori_loop(0, chunk, zrow_body, None)

        def dz_body(i, _):
            dzero[pl.ds(i * 16, 16)] = zeros16
            return _
        lax.fori_loop(0, rows_per_tile // 16, dz_body, None)

        def ones_body(i, _):
            ones_v[pl.ds(i * 16, 16)] = ones16
            return _
        lax.fori_loop(0, chunk // 16, ones_body, None)

        for k in range(zcopies):
            pltpu.sync_copy(rows, agg_sh.at[pl.ds(base_r + k * chunk, chunk)])
        pltpu.sync_copy(dzero, deg_sh.at[pl.ds(base_r, rows_per_tile)])
        plsc.subcore_barrier()

        ebase = wid * ew

        def edge_body(j, _):
            off = pl.multiple_of(ebase + j * chunk, 8)
            pltpu.sync_copy(src_hbm.at[pl.ds(off, chunk)], src_idx)
            pltpu.sync_copy(dst_hbm.at[pl.ds(off, chunk)], dst_idx)
            pltpu.async_copy(feat_hbm.at[src_idx], rows, sem).wait()
            pltpu.sync_copy(rows, agg_sh.at[dst_idx], add=True)
            pltpu.sync_copy(ones_v, deg_sh.at[dst_idx], add=True)
            return _
        lax.fori_loop(0, n_iter, edge_body, None)

        plsc.subcore_barrier()

        out_r = pl.multiple_of(c * n_pad + base_r, 8)
        pltpu.sync_copy(agg_sh.at[pl.ds(base_r, rows_per_tile)],
                        agg_out.at[pl.ds(out_r, rows_per_tile)])
        pltpu.sync_copy(deg_sh.at[pl.ds(base_r, rows_per_tile)],
                        deg_out.at[pl.ds(out_r, rows_per_tile)])

    return sc_kernel(feat, src, dst)


def _tc_combine(feat, agg_parts, deg_parts, w_self, w_neigh, b, blk):
    n, d = feat.shape
    d_out = w_self.shape[1]
    grid = n // blk
    deg3 = deg_parts[:, :, None]
    b2 = b[None, :]

    def body(feat_ref, agg_ref, deg_ref, ws_ref, wn_ref, b_ref, out_ref):
        agg = agg_ref[0] + agg_ref[1]
        deg = jnp.maximum(deg_ref[0] + deg_ref[1], 1.0)
        h = agg / deg
        out_ref[...] = (
            jnp.dot(feat_ref[...], ws_ref[...],
                    preferred_element_type=jnp.float32)
            + jnp.dot(h, wn_ref[...], preferred_element_type=jnp.float32)
            + b_ref[...]
        )

    return pl.pallas_call(
        body,
        grid=(grid,),
        in_specs=[
            pl.BlockSpec((blk, d), lambda i: (i, 0)),
            pl.BlockSpec((NC, blk, d), lambda i: (0, i, 0)),
            pl.BlockSpec((NC, blk, 1), lambda i: (0, i, 0)),
            pl.BlockSpec((d, d_out), lambda i: (0, 0)),
            pl.BlockSpec((d, d_out), lambda i: (0, 0)),
            pl.BlockSpec((1, d_out), lambda i: (0, 0)),
        ],
        out_specs=pl.BlockSpec((blk, d_out), lambda i: (i, 0)),
        out_shape=jax.ShapeDtypeStruct((n, d_out), jnp.float32),
    )(feat, agg_parts, deg3, w_self, w_neigh, b2)


def kernel(feat, edge_index, W_self, W_neigh, b):
    n, d = feat.shape
    chunk = 80
    n_pad = -(-n // (NS * chunk)) * (NS * chunk)
    src = edge_index[0]
    dst = edge_index[1]
    agg_flat, deg_flat = _sc_aggregate(feat, src, dst, n_pad, chunk=chunk)
    agg_parts = agg_flat.reshape(NC, n_pad, d)
    deg_parts = deg_flat.reshape(NC, n_pad)
    out = _tc_combine(feat, agg_parts, deg_parts, W_self, W_neigh, b, blk=2000)
    return out
